# baseline (speedup 1.0000x reference)
import functools

import jax
import jax.numpy as jnp
from jax import lax
from jax.experimental import pallas as pl
from jax.experimental.pallas import tpu as pltpu
from jax.experimental.pallas import tpu_sc as plsc

D = 64
DP = 128
SCALE = 8.0
L = 16


def kernel(x, emb):
    B, S = x.shape
    N = B * S
    info = plsc.get_sparse_core_info()
    NC, NS = info.num_cores, info.num_subcores
    NW = NC * NS
    assert B == NW * DP
    assert S % 2 == 0

    mesh = plsc.VectorSubcoreMesh(core_axis_name="c", subcore_axis_name="s")

    @functools.partial(
        pl.kernel,
        mesh=mesh,
        out_type=jax.ShapeDtypeStruct((S, D // 8, B // DP, 8, DP),
                                      jnp.float32),
        scratch_types=[
            pltpu.VMEM((DP,), jnp.int32),
            pltpu.VMEM((DP,), jnp.int32),
            pltpu.VMEM((DP, D), jnp.float32),
            pltpu.VMEM((DP, D), jnp.float32),
            pltpu.VMEM((D // 8, 8, DP), jnp.float32),
            pltpu.VMEM((D // 8, 8, DP), jnp.float32),
            pltpu.SemaphoreType.DMA,
            pltpu.SemaphoreType.DMA,
            pltpu.SemaphoreType.DMA,
            pltpu.SemaphoreType.DMA,
            pltpu.SemaphoreType.DMA,
            pltpu.SemaphoreType.DMA,
        ],
        compiler_params=pltpu.CompilerParams(use_tc_tiling_on_sc=False,
                                             needs_layout_passes=False),
    )
    def emb_kernel(xt_hbm, table_hbm, out_hbm,
                   ix0, ix1, g0, g1, o0, o1,
                   isem0, isem1, gsem0, gsem1, osem0, osem1):
        wid = lax.axis_index("s") * NC + lax.axis_index("c")
        b0 = wid * DP
        idx = (ix0, ix1)
        G = (g0, g1)
        O = (o0, o1)
        isem = (isem0, isem1)
        gsem = (gsem0, gsem1)
        osem = (osem0, osem1)

        iota = lax.iota(jnp.int32, L)

        def start_idx(s, b):
            pltpu.async_copy(xt_hbm.at[s, pl.ds(b0, DP)], idx[b], isem[b])

        def wait_idx(b):
            pltpu.make_async_copy(xt_hbm.at[0, pl.ds(b0, DP)], idx[b],
                                  isem[b]).wait()

        def start_gather(b):
            pltpu.async_copy(table_hbm.at[idx[b]], G[b], gsem[b])

        def wait_gather(b):
            pltpu.make_async_copy(table_hbm.at[idx[b]], G[b], gsem[b]).wait()

        def transpose_block(b):
            gb, ob = G[b], O[b]
            rows = [iota + (t * L) for t in range(DP // L)]
            zero = iota * 0

            for d in range(D):
                cols = zero + d
                for t in range(DP // L):
                    vals = plsc.load_gather(gb, [rows[t], cols])
                    ob[d >> 3, d & 7, pl.ds(t * L, L)] = vals * SCALE

        def start_out(s, b):
            pltpu.async_copy(O[b], out_hbm.at[s, :, wid, :, :], osem[b])

        def wait_out(b):
            pltpu.make_async_copy(O[b], out_hbm.at[0, :, wid, :, :],
                                  osem[b]).wait()

        start_idx(0, 0)
        wait_idx(0)
        start_gather(0)
        start_idx(1, 1)

        def pair_body(g, carry):
            s = 2 * g
            wait_gather(0)
            wait_idx(1)
            start_gather(1)

            @pl.when(s + 2 < S)
            def _():
                start_idx(s + 2, 0)

            @pl.when(g >= 1)
            def _():
                wait_out(0)

            transpose_block(0)
            start_out(s, 0)

            wait_gather(1)

            @pl.when(s + 2 < S)
            def _():
                wait_idx(0)
                start_gather(0)

            @pl.when(s + 3 < S)
            def _():
                start_idx(s + 3, 1)

            @pl.when(g >= 1)
            def _():
                wait_out(1)

            transpose_block(1)
            start_out(s + 1, 1)
            return carry

        lax.fori_loop(0, S // 2, pair_body, 0, unroll=False)

        wait_out(0)
        wait_out(1)

    xt = x.T
    y = emb_kernel(xt, emb)
    return y.transpose(2, 4, 0, 1, 3).reshape(B, S, D)


# speedup vs baseline: 1.3360x; 1.3360x over previous
import functools

import jax
import jax.numpy as jnp
from jax import lax
from jax.experimental import pallas as pl
from jax.experimental.pallas import tpu as pltpu
from jax.experimental.pallas import tpu_sc as plsc

D = 64
DP = 128
SCALE = 8.0
L = 16


def kernel(x, emb):
    B, S = x.shape
    N = B * S
    info = plsc.get_sparse_core_info()
    NC, NS = info.num_cores, info.num_subcores
    NW = NC * NS
    assert B == NW * DP
    assert S % 2 == 0

    mesh = plsc.VectorSubcoreMesh(core_axis_name="c", subcore_axis_name="s")

    @functools.partial(
        pl.kernel,
        mesh=mesh,
        out_type=jax.ShapeDtypeStruct((S, D // 8, B // DP, 8, DP),
                                      jnp.float32),
        scratch_types=[
            pltpu.VMEM((DP,), jnp.int32),
            pltpu.VMEM((DP,), jnp.int32),
            pltpu.VMEM((DP, D), jnp.float32),
            pltpu.VMEM((DP, D), jnp.float32),
            pltpu.VMEM((D // 8, 8, DP), jnp.float32),
            pltpu.VMEM((D // 8, 8, DP), jnp.float32),
            pltpu.SemaphoreType.DMA,
            pltpu.SemaphoreType.DMA,
            pltpu.SemaphoreType.DMA,
            pltpu.SemaphoreType.DMA,
            pltpu.SemaphoreType.DMA,
            pltpu.SemaphoreType.DMA,
        ],
        compiler_params=pltpu.CompilerParams(use_tc_tiling_on_sc=False,
                                             needs_layout_passes=False),
    )
    def emb_kernel(xt_hbm, table_hbm, out_hbm,
                   ix0, ix1, g0, g1, o0, o1,
                   isem0, isem1, gsem0, gsem1, osem0, osem1):
        wid = lax.axis_index("s") * NC + lax.axis_index("c")
        b0 = wid * DP
        idx = (ix0, ix1)
        G = (g0, g1)
        O = (o0, o1)
        isem = (isem0, isem1)
        gsem = (gsem0, gsem1)
        osem = (osem0, osem1)

        iota = lax.iota(jnp.int32, L)
        zero = iota * 0
        dhi = [lax.shift_right_logical(iota + 16 * q, 3)
               for q in range(D // L)]
        dlo = [lax.bitwise_and(iota + 16 * q, 7) for q in range(D // L)]

        def start_idx(s, b):
            pltpu.async_copy(xt_hbm.at[s, pl.ds(b0, DP)], idx[b], isem[b])

        def wait_idx(b):
            pltpu.make_async_copy(xt_hbm.at[0, pl.ds(b0, DP)], idx[b],
                                  isem[b]).wait()

        def start_gather(b):
            pltpu.async_copy(table_hbm.at[idx[b]], G[b], gsem[b])

        def wait_gather(b):
            pltpu.make_async_copy(table_hbm.at[idx[b]], G[b], gsem[b]).wait()

        def transpose_block(b):
            gb, ob = G[b], O[b]

            def jbody(j, carry):
                colj = zero + j
                vals = [gb[j, pl.ds(L * q, L)] * SCALE
                        for q in range(D // L)]
                for q in range(D // L):
                    plsc.store_scatter(ob, [dhi[q], dlo[q], colj], vals[q])
                return carry

            lax.fori_loop(0, DP, jbody, 0, unroll=2)

        def start_out(s, b):
            pltpu.async_copy(O[b], out_hbm.at[s, :, wid, :, :], osem[b])

        def wait_out(b):
            pltpu.make_async_copy(O[b], out_hbm.at[0, :, wid, :, :],
                                  osem[b]).wait()

        start_idx(0, 0)
        wait_idx(0)
        start_gather(0)
        start_idx(1, 1)

        def pair_body(g, carry):
            s = 2 * g
            wait_gather(0)
            wait_idx(1)
            start_gather(1)

            @pl.when(s + 2 < S)
            def _():
                start_idx(s + 2, 0)

            @pl.when(g >= 1)
            def _():
                wait_out(0)

            transpose_block(0)
            start_out(s, 0)

            wait_gather(1)

            @pl.when(s + 2 < S)
            def _():
                wait_idx(0)
                start_gather(0)

            @pl.when(s + 3 < S)
            def _():
                start_idx(s + 3, 1)

            @pl.when(g >= 1)
            def _():
                wait_out(1)

            transpose_block(1)
            start_out(s + 1, 1)
            return carry

        lax.fori_loop(0, S // 2, pair_body, 0, unroll=False)

        wait_out(0)
        wait_out(1)

    xt = x.T
    y = emb_kernel(xt, emb)
    return y.transpose(2, 4, 0, 1, 3).reshape(B, S, D)


# speedup vs baseline: 2.3236x; 1.7392x over previous
import functools

import jax
import jax.numpy as jnp
from jax import lax
from jax.experimental import pallas as pl
from jax.experimental.pallas import tpu as pltpu
from jax.experimental.pallas import tpu_sc as plsc

D = 64
DP = 128
SB = 2
DPP = DP + 1
SCALE = 8.0
L = 16


def kernel(x, emb):
    B, S = x.shape
    info = plsc.get_sparse_core_info()
    NC, NS = info.num_cores, info.num_subcores
    NW = NC * NS
    assert B == NW * DP
    NB = S // SB
    assert S % (2 * SB) == 0

    mesh = plsc.VectorSubcoreMesh(core_axis_name="c", subcore_axis_name="s")

    @functools.partial(
        pl.kernel,
        mesh=mesh,
        out_type=jax.ShapeDtypeStruct((S, D // 8, B // DP, 8, DP),
                                      jnp.float32),
        scratch_types=[
            pltpu.VMEM((SB * DP,), jnp.int32),
            pltpu.VMEM((SB * DP,), jnp.int32),
            pltpu.VMEM((SB * DP, D), jnp.float32),
            pltpu.VMEM((SB * DP, D), jnp.float32),
            pltpu.VMEM((SB, D // 8, 8, DPP), jnp.float32),
            pltpu.VMEM((SB, D // 8, 8, DPP), jnp.float32),
            pltpu.SemaphoreType.DMA,
            pltpu.SemaphoreType.DMA,
            pltpu.SemaphoreType.DMA,
            pltpu.SemaphoreType.DMA,
            pltpu.SemaphoreType.DMA,
            pltpu.SemaphoreType.DMA,
        ],
        compiler_params=pltpu.CompilerParams(use_tc_tiling_on_sc=False,
                                             needs_layout_passes=False),
    )
    def emb_kernel(xt_hbm, table_hbm, out_hbm,
                   ix0, ix1, g0, g1, o0, o1,
                   isem0, isem1, gsem0, gsem1, osem0, osem1):
        wid = lax.axis_index("s") * NC + lax.axis_index("c")
        b0 = wid * DP
        idx = (ix0, ix1)
        G = (g0, g1)
        O = (o0, o1)
        isem = (isem0, isem1)
        gsem = (gsem0, gsem1)
        osem = (osem0, osem1)

        iota = lax.iota(jnp.int32, L)
        zero = iota * 0
        dhi = [lax.shift_right_logical(iota + L * q, 3)
               for q in range(D // L)]
        dlo = [lax.bitwise_and(iota + L * q, 7) for q in range(D // L)]

        def start_idx(k, b):
            s = SB * k
            for u in range(SB):
                pltpu.async_copy(xt_hbm.at[s + u, pl.ds(b0, DP)],
                                 idx[b].at[pl.ds(u * DP, DP)], isem[b])

        def wait_idx(b):
            for u in range(SB):
                pltpu.make_async_copy(xt_hbm.at[0, pl.ds(b0, DP)],
                                      idx[b].at[pl.ds(u * DP, DP)],
                                      isem[b]).wait()

        def start_gather(b):
            pltpu.async_copy(table_hbm.at[idx[b]], G[b], gsem[b])

        def wait_gather(b):
            pltpu.make_async_copy(table_hbm.at[idx[b]], G[b], gsem[b]).wait()

        def transpose_block(b):
            gb, ob = G[b], O[b]

            def jbody(j, carry):
                shi = zero + (j >> 7)
                colj = zero + (j & (DP - 1))
                vals = [gb[j, pl.ds(L * q, L)] * SCALE
                        for q in range(D // L)]
                for q in range(D // L):
                    plsc.store_scatter(ob, [shi, dhi[q], dlo[q], colj],
                                       vals[q])
                return carry

            lax.fori_loop(0, SB * DP, jbody, 0, unroll=2)

        def start_out(k, b):
            s = SB * k
            pltpu.async_copy(O[b].at[:, :, :, pl.ds(0, DP)],
                             out_hbm.at[pl.ds(s, SB), :, wid, :, :], osem[b])

        def wait_out(b):
            pltpu.make_async_copy(O[b].at[:, :, :, pl.ds(0, DP)],
                                  out_hbm.at[pl.ds(0, SB), :, wid, :, :],
                                  osem[b]).wait()

        start_idx(0, 0)
        wait_idx(0)
        start_gather(0)
        start_idx(1, 1)

        def pair_body(g, carry):
            k = 2 * g
            wait_gather(0)
            wait_idx(1)
            start_gather(1)

            @pl.when(k + 2 < NB)
            def _():
                start_idx(k + 2, 0)

            @pl.when(g >= 1)
            def _():
                wait_out(0)

            transpose_block(0)
            start_out(k, 0)

            wait_gather(1)

            @pl.when(k + 2 < NB)
            def _():
                wait_idx(0)
                start_gather(0)

            @pl.when(k + 3 < NB)
            def _():
                start_idx(k + 3, 1)

            @pl.when(g >= 1)
            def _():
                wait_out(1)

            transpose_block(1)
            start_out(k + 1, 1)
            return carry

        lax.fori_loop(0, NB // 2, pair_body, 0, unroll=False)

        wait_out(0)
        wait_out(1)

    xt = x.T
    y = emb_kernel(xt, emb)
    return y.transpose(2, 4, 0, 1, 3).reshape(B, S, D)
